# baseline (speedup 1.0000x reference)
import functools

import jax
import jax.numpy as jnp
from jax import lax
from jax.experimental import pallas as pl
from jax.experimental.pallas import tpu as pltpu

NUM_EMB = 8192
DIM = 256
ROWS = 8192
BLOCK = 256
NBLK = ROWS // BLOCK


def _vq_kernel(x_ref, w_ref, enc_ref, qst_ref, loss_ref, perp_ref,
               sw_ref, counts_ref, loss_acc_ref):
    i = pl.program_id(0) * (ROWS // BLOCK // 8) + pl.program_id(1)

    @pl.when(i == 0)
    def _init():
        w = w_ref[...]
        ones = jnp.ones((1, DIM), jnp.float32)
        sw_ref[...] = lax.dot_general(
            ones, w * w, (((1,), (1,)), ((), ())),
            preferred_element_type=jnp.float32)
        counts_ref[...] = jnp.zeros((1, NUM_EMB), jnp.float32)
        loss_acc_ref[0, 0] = 0.0

    x = jnp.transpose(x_ref[0], (1, 0))
    w = w_ref[...]
    sx = jnp.sum(x * x, axis=1, keepdims=True)
    mm = lax.dot_general(x, w, (((1,), (1,)), ((), ())),
                         preferred_element_type=jnp.float32)
    d = (sx + sw_ref[...]) - 2.0 * mm
    dmin = jnp.min(d, axis=1, keepdims=True)
    iota = lax.broadcasted_iota(jnp.int32, (BLOCK, NUM_EMB), 1)
    idx = jnp.min(jnp.where(d == dmin, iota, NUM_EMB), axis=1,
                  keepdims=True)
    enc = (iota == idx).astype(jnp.float32)
    enc_ref[...] = enc

    q = lax.dot_general(enc, w, (((1,), (0,)), ((), ())),
                        preferred_element_type=jnp.float32)
    qst_ref[0] = jnp.transpose(x - (q - x), (1, 0))

    counts_ref[...] += jnp.sum(enc, axis=0, keepdims=True)
    loss_acc_ref[0, 0] += jnp.sum(dmin)

    @pl.when(i == NBLK - 1)
    def _finalize():
        loss_ref[0, 0] = 1.25 * loss_acc_ref[0, 0] / (ROWS * DIM)
        p = counts_ref[...] * (1.0 / ROWS)
        perp_ref[0, 0] = jnp.exp(-jnp.sum(p * jnp.log(p + 1e-10)))


PBLK = ROWS // BLOCK // 8


@functools.partial(jax.jit)
def kernel(inputs, weight):
    x_cp = inputs.reshape(8, DIM, 1024)

    enc, qst, loss, perp = pl.pallas_call(
        _vq_kernel,
        grid=(8, PBLK),
        in_specs=[
            pl.BlockSpec((1, DIM, BLOCK), lambda b, p: (b, 0, p)),
            pl.BlockSpec((NUM_EMB, DIM), lambda b, p: (0, 0)),
        ],
        out_specs=[
            pl.BlockSpec((BLOCK, NUM_EMB), lambda b, p: (b * PBLK + p, 0)),
            pl.BlockSpec((1, DIM, BLOCK), lambda b, p: (b, 0, p)),
            pl.BlockSpec(memory_space=pltpu.SMEM),
            pl.BlockSpec(memory_space=pltpu.SMEM),
        ],
        out_shape=[
            jax.ShapeDtypeStruct((ROWS, NUM_EMB), jnp.float32),
            jax.ShapeDtypeStruct((8, DIM, 1024), jnp.float32),
            jax.ShapeDtypeStruct((1, 1), jnp.float32),
            jax.ShapeDtypeStruct((1, 1), jnp.float32),
        ],
        scratch_shapes=[
            pltpu.VMEM((1, NUM_EMB), jnp.float32),
            pltpu.VMEM((1, NUM_EMB), jnp.float32),
            pltpu.SMEM((1, 1), jnp.float32),
        ],
    )(x_cp, weight)

    quantized_st = qst.reshape(8, DIM, 32, 32)
    encodings = enc.reshape(ROWS, 1, NUM_EMB)
    return (loss[0, 0], quantized_st, perp[0, 0], encodings)


# speedup vs baseline: 1.0534x; 1.0534x over previous
import functools

import jax
import jax.numpy as jnp
from jax import lax
from jax.experimental import pallas as pl
from jax.experimental.pallas import tpu as pltpu

NUM_EMB = 8192
DIM = 256
ROWS = 8192
BLOCK = 256
NBLK = ROWS // BLOCK


def _vq_kernel(x_ref, w_ref, enc_ref, qst_ref, loss_ref, perp_ref,
               sw_ref, counts_ref, wbf_ref, loss_acc_ref):
    i = pl.program_id(0)

    @pl.when(i == 0)
    def _init():
        w = w_ref[...]
        ones = jnp.ones((1, DIM), jnp.float32)
        sw_ref[...] = lax.dot_general(
            ones, w * w, (((1,), (1,)), ((), ())),
            preferred_element_type=jnp.float32)
        counts_ref[...] = jnp.zeros((1, NUM_EMB), jnp.float32)
        wbf_ref[...] = w.astype(jnp.bfloat16)
        loss_acc_ref[0, 0] = 0.0

    x = x_ref[...]
    sx = jnp.sum(x * x, axis=1, keepdims=True)
    mm = lax.dot_general(x, w_ref[...], (((1,), (1,)), ((), ())),
                         preferred_element_type=jnp.float32)
    d = (sx + sw_ref[...]) - 2.0 * mm
    dmin = jnp.min(d, axis=1, keepdims=True)
    iota = lax.broadcasted_iota(jnp.int32, (BLOCK, NUM_EMB), 1)
    idx = jnp.min(jnp.where(d == dmin, iota, NUM_EMB), axis=1,
                  keepdims=True)
    enc = (iota == idx).astype(jnp.float32)
    enc_ref[...] = enc

    enc_bf = enc.astype(jnp.bfloat16)
    q = lax.dot_general(enc_bf, wbf_ref[...], (((1,), (0,)), ((), ())),
                        preferred_element_type=jnp.float32)
    qst_ref[...] = x - (q - x)

    ones_bf = jnp.ones((1, BLOCK), jnp.bfloat16)
    counts_ref[...] += lax.dot_general(
        ones_bf, enc_bf, (((1,), (0,)), ((), ())),
        preferred_element_type=jnp.float32)
    loss_acc_ref[0, 0] += jnp.sum(dmin)

    @pl.when(i == NBLK - 1)
    def _finalize():
        loss_ref[0, 0] = 1.25 * loss_acc_ref[0, 0] / (ROWS * DIM)
        p = counts_ref[...] * (1.0 / ROWS)
        perp_ref[0, 0] = jnp.exp(-jnp.sum(p * jnp.log(p + 1e-10)))


@functools.partial(jax.jit)
def kernel(inputs, weight):
    x_flat = jnp.transpose(inputs, (0, 2, 3, 1)).reshape(ROWS, DIM)

    enc, qst, loss, perp = pl.pallas_call(
        _vq_kernel,
        grid=(NBLK,),
        in_specs=[
            pl.BlockSpec((BLOCK, DIM), lambda i: (i, 0)),
            pl.BlockSpec((NUM_EMB, DIM), lambda i: (0, 0)),
        ],
        out_specs=[
            pl.BlockSpec((BLOCK, NUM_EMB), lambda i: (i, 0)),
            pl.BlockSpec((BLOCK, DIM), lambda i: (i, 0)),
            pl.BlockSpec(memory_space=pltpu.SMEM),
            pl.BlockSpec(memory_space=pltpu.SMEM),
        ],
        out_shape=[
            jax.ShapeDtypeStruct((ROWS, NUM_EMB), jnp.float32),
            jax.ShapeDtypeStruct((ROWS, DIM), jnp.float32),
            jax.ShapeDtypeStruct((1, 1), jnp.float32),
            jax.ShapeDtypeStruct((1, 1), jnp.float32),
        ],
        scratch_shapes=[
            pltpu.VMEM((1, NUM_EMB), jnp.float32),
            pltpu.VMEM((1, NUM_EMB), jnp.float32),
            pltpu.VMEM((NUM_EMB, DIM), jnp.bfloat16),
            pltpu.SMEM((1, 1), jnp.float32),
        ],
    )(x_flat, weight)

    quantized_st = jnp.transpose(
        qst.reshape(8, 32, 32, DIM), (0, 3, 1, 2))
    encodings = enc.reshape(ROWS, 1, NUM_EMB)
    return (loss[0, 0], quantized_st, perp[0, 0], encodings)
